# Initial kernel scaffold; baseline (speedup 1.0000x reference)
import jax
import jax.numpy as jnp
from jax.experimental import pallas as pl


def kernel(edge_index, edge_type, encoder_output, seed_entities, sample_mask, nodeID2node, entity_state, state, entity_table, rel_table, W_ih, W_hh, b_ih, b_hh, sub_W, sub_b, obj_W, obj_b):
    raise NotImplementedError("write your pallas kernel here")


# speedup vs baseline: 70.8062x; 70.8062x over previous
import functools

import jax
import jax.numpy as jnp
from jax import lax
from jax.experimental import pallas as pl
from jax.experimental.pallas import tpu as pltpu
from jax.experimental.pallas import tpu_sc as plsc

N = 10000
E = 320000
D = 128
NUM_ENT = 100000
NUM_REL_PAD = 512

NW = 32
CHUNK = E // NW
BPB = 25
NBLK = CHUNK // (16 * BPB)
SLOTCAP = CHUNK
ROWS = (N + NW - 1) // NW
WPAD = 320

def _al(x, m=8):
    return pl.multiple_of(x, m)


_mesh = plsc.VectorSubcoreMesh(core_axis_name="c", subcore_axis_name="s")


def _dense_tc(sample_mask, encoder_output, rel_pad, W_ih, W_hh, b_ih, b_hh,
              sub_W, sub_b, obj_W, obj_b):
    def body(sm_ref, enc_ref, rel_ref, wih_ref, whh_ref, bih_ref, bhh_ref,
             subw_ref, subb_ref, objw_ref, objb_ref, obj_out, sub_out):
        def mmT(x, w):
            return lax.dot_general(x, w, (((1,), (1,)), ((), ())),
                                   preferred_element_type=jnp.float32)

        sub_h = jnp.tanh(mmT(sm_ref[...], subw_ref[...]) + subb_ref[...])

        gi0 = mmT(enc_ref[...], wih_ref[...]) + bih_ref[...]
        gh0 = mmT(sub_h, whh_ref[...]) + bhh_ref[...]
        r_ = jax.nn.sigmoid(gi0[:, 0:D] + gh0[:, 0:D])
        z_ = jax.nn.sigmoid(gi0[:, D:2 * D] + gh0[:, D:2 * D])
        n_ = jnp.tanh(gi0[:, 2 * D:3 * D] + r_ * gh0[:, 2 * D:3 * D])
        r0 = (1.0 - z_) * n_ + z_ * sub_h

        rel = rel_ref[...]
        gi = mmT(r0, wih_ref[...]) + bih_ref[...]
        gh = mmT(rel, whh_ref[...]) + bhh_ref[...]
        r = jax.nn.sigmoid(gi[:, 0:D] + gh[:, 0:D])
        z = jax.nn.sigmoid(gi[:, D:2 * D] + gh[:, D:2 * D])
        n = jnp.tanh(gi[:, 2 * D:3 * D] + r * gh[:, 2 * D:3 * D])
        rj = (1.0 - z) * n + z * rel
        obj_out[...] = jnp.tanh(mmT(rj, objw_ref[...]) + objb_ref[...])
        sub_out[...] = sub_h

    return pl.pallas_call(
        body,
        out_shape=(
            jax.ShapeDtypeStruct((NUM_REL_PAD, D), jnp.float32),
            jax.ShapeDtypeStruct((1, D), jnp.float32),
        ),
    )(sample_mask, encoder_output, rel_pad, W_ih, W_hh, b_ih, b_hh,
      sub_W, sub_b, obj_W, obj_b)


@functools.partial(
    pl.kernel,
    mesh=_mesh,
    out_type=(
        jax.ShapeDtypeStruct((NW * SLOTCAP * 16,), jnp.int32),
        jax.ShapeDtypeStruct((NW * 16,), jnp.int32),
    ),
    scratch_types=[
        pltpu.VMEM((CHUNK,), jnp.int32),
        pltpu.VMEM((CHUNK,), jnp.int32),
        pltpu.VMEM((16,), jnp.int32),
        pltpu.VMEM((32,), jnp.int32),
        pltpu.VMEM((32,), jnp.int32),
        pltpu.VMEM((16,), jnp.int32),
        pltpu.VMEM((16,), jnp.int32),
        pltpu.SMEM((8,), jnp.int32),
    ],
)
def _sc_filter(heads_hbm, tails_hbm, seed_hbm, slots_hbm, cnt_hbm,
               h_v, t_v, seed_v, sa_v, sb_v, slot_v, cb_v, sm):
    w = lax.axis_index("s") * 2 + lax.axis_index("c")
    base_e = w * CHUNK
    pltpu.sync_copy(heads_hbm.at[pl.ds(_al(base_e), CHUNK)], h_v)
    pltpu.sync_copy(tails_hbm.at[pl.ds(_al(base_e), CHUNK)], t_v)
    pltpu.sync_copy(seed_hbm, seed_v)
    seed_s = seed_v[...][0]
    sm[0] = jnp.int32(0)
    iota = lax.iota(jnp.int32, 16)

    def block(b, carry):
        sv = jnp.zeros((16,), jnp.int32) + seed_s
        acc = jnp.zeros((16,), jnp.int32) + (b * 0)
        for vi in range(BPB):
            xv = h_v[pl.ds((b * BPB + vi) * 16, 16)]
            acc = acc + jnp.where(xv == sv, jnp.int32(1 << vi), jnp.int32(0))
        sa_v[pl.ds(0, 16)] = acc
        for sh in (8, 4, 2, 1):
            sa_v[pl.ds(0, 16)] = sa_v[pl.ds(0, 16)] + sa_v[pl.ds(sh, 16)]
        tot = sa_v[pl.ds(0, 16)][0]

        @pl.when(tot != 0)
        def _():
            sa_v[pl.ds(0, 16)] = acc

            def lanes(j, c2):
                bits = sa_v[pl.ds(j, 16)][0]

                @pl.when(bits != 0)
                def _():
                    def vbits(vi, c3):
                        @pl.when(((bits >> vi) & 1) != 0)
                        def _():
                            vec = b * BPB + vi
                            e_idx = base_e + vec * 16 + j
                            sb_v[pl.ds(0, 16)] = t_v[pl.ds(vec * 16, 16)]
                            tval = sb_v[pl.ds(j, 16)][0]
                            off = sm[0]
                            slot_v[...] = jnp.where(
                                iota < 8,
                                jnp.zeros((16,), jnp.int32) + e_idx,
                                jnp.zeros((16,), jnp.int32) + tval)
                            pltpu.sync_copy(
                                slot_v,
                                slots_hbm.at[pl.ds(_al((w * SLOTCAP + off) * 16, 16), 16)])
                            sm[0] = off + jnp.int32(1)
                        return c3

                    lax.fori_loop(0, BPB, vbits, jnp.int32(0))
                return c2

            lax.fori_loop(0, 16, lanes, jnp.int32(0))
        return carry

    lax.fori_loop(0, NBLK, block, jnp.int32(0))
    cb_v[...] = jnp.zeros((16,), jnp.int32) + sm[0]
    pltpu.sync_copy(cb_v, cnt_hbm.at[pl.ds(_al(w * 16, 16), 16)])


@functools.partial(
    pl.kernel,
    mesh=_mesh,
    out_type=jax.ShapeDtypeStruct((N * D,), jnp.float32),
    scratch_types=[
        pltpu.VMEM((NW * 16,), jnp.int32),
        pltpu.VMEM((16,), jnp.int32),
        pltpu.VMEM((128,), jnp.float32),
        pltpu.VMEM((WPAD * 16,), jnp.int32),
        pltpu.VMEM((144,), jnp.int32),
        pltpu.VMEM((32,), jnp.int32),
        pltpu.VMEM((128,), jnp.float32),
        pltpu.VMEM((8192,), jnp.float32),
        pltpu.SMEM((8,), jnp.int32),
    ],
)
def _sc_apply(slots_hbm, cnt_hbm, seed_hbm, sub_hbm, etype_hbm, n2n_hbm,
              estate_hbm, etab_hbm, objrel_hbm, out_hbm,
              cnts_v, seed_v, sub_v, win_v, sblk_v, g_v, row_v, zb_v, sm):
    w = lax.axis_index("s") * 2 + lax.axis_index("c")
    base = w * ROWS
    nrows = jnp.minimum(ROWS, N - base)
    pltpu.sync_copy(cnt_hbm, cnts_v)
    pltpu.sync_copy(seed_hbm, seed_v)
    pltpu.sync_copy(sub_hbm, sub_v)
    seed_s = seed_v[...][0]

    def wini(i, c):
        win_v[pl.ds(i * 16, 16)] = jnp.zeros((16,), jnp.int32) + (i * 0 - 1)
        return c

    lax.fori_loop(0, WPAD, wini, jnp.int32(0))

    def zini(i, c):
        zb_v[pl.ds(i * 16, 16)] = jnp.zeros((16,), jnp.float32) * (i * 1.0)
        return c

    lax.fori_loop(0, 512, zini, jnp.int32(0))

    def region(r, c):
        cntr = cnts_v[pl.ds(r * 16, 16)][0]

        def qblk(qb, c2):
            pltpu.sync_copy(
                slots_hbm.at[pl.ds(_al((r * SLOTCAP + qb * 8) * 16, 16), 128)],
                sblk_v.at[pl.ds(0, 128)])
            for s in range(8):
                q = qb * 8 + s

                @pl.when(q < cntr)
                def _():
                    e_s = sblk_v[pl.ds(s * 16, 16)][0]
                    t_s = sblk_v[pl.ds(s * 16 + 8, 16)][0]

                    @pl.when((t_s >= base) & (t_s < base + nrows))
                    def _():
                        win_v[pl.ds((t_s - base) * 16, 16)] = (
                            jnp.zeros((16,), jnp.int32) + e_s)
            return c2

        lax.fori_loop(0, (cntr + 7) >> 3, qblk, jnp.int32(0))
        return c

    lax.fori_loop(0, NW, region, jnp.int32(0))

    def zc(c, c2):
        start = jnp.minimum(base + c * 64, base + nrows - 64)
        pltpu.sync_copy(zb_v, out_hbm.at[pl.ds(_al(start * D, D), 64 * D)])
        return c2

    lax.fori_loop(0, (nrows + 63) >> 6, zc, jnp.int32(0))

    @pl.when((seed_s >= base) & (seed_s < base + nrows))
    def _():
        pltpu.sync_copy(sub_v, out_hbm.at[pl.ds(_al(seed_s * D, D), D)])

    def sweep(i, c):
        e = win_v[pl.ds(i * 16, 16)][0]

        @pl.when(e >= 0)
        def _():
            t = base + i
            e8 = (e >> 3) << 3
            pltpu.sync_copy(etype_hbm.at[pl.ds(_al(e8), 16)], g_v.at[pl.ds(0, 16)])
            ty = g_v[pl.ds(e - e8, 16)][0]
            t8 = (t >> 3) << 3
            pltpu.sync_copy(n2n_hbm.at[pl.ds(_al(t8), 16)], g_v.at[pl.ds(0, 16)])
            orig = g_v[pl.ds(t - t8, 16)][0]
            o8 = (orig >> 3) << 3
            pltpu.sync_copy(estate_hbm.at[pl.ds(_al(o8), 16)], g_v.at[pl.ds(0, 16)])
            ts = g_v[pl.ds(orig - o8, 16)][0]

            @pl.when(ts == 1)
            def _():
                pltpu.sync_copy(etab_hbm.at[pl.ds(_al(orig * D, D), D)], row_v)

            @pl.when(ts != 1)
            def _():
                pltpu.sync_copy(objrel_hbm.at[pl.ds(_al(ty * D, D), D)], row_v)

            pltpu.sync_copy(row_v, out_hbm.at[pl.ds(_al(t * D, D), D)])
        return c

    lax.fori_loop(0, nrows, sweep, jnp.int32(0))


def kernel(edge_index, edge_type, encoder_output, seed_entities, sample_mask,
           nodeID2node, entity_state, state, entity_table, rel_table,
           W_ih, W_hh, b_ih, b_hh, sub_W, sub_b, obj_W, obj_b):
    heads = edge_index[0].astype(jnp.int32)
    tails = edge_index[1].astype(jnp.int32)
    etype_pad = jnp.pad(edge_type.astype(jnp.int32), (0, 16))
    n2n_pad = jnp.pad(nodeID2node.astype(jnp.int32), (0, 16))
    estate_pad = jnp.pad(entity_state.astype(jnp.int32), (0, 16))
    seed16 = jnp.full((16,), seed_entities[0], jnp.int32)
    rel_pad = jnp.zeros((NUM_REL_PAD, D), jnp.float32).at[:rel_table.shape[0]].set(rel_table)

    objrel, sub_emb = _dense_tc(
        sample_mask, encoder_output, rel_pad, W_ih, W_hh,
        b_ih.reshape(1, -1), b_hh.reshape(1, -1),
        sub_W, sub_b.reshape(1, -1), obj_W, obj_b.reshape(1, -1))

    slots, cnts = _sc_filter(heads, tails, seed16)

    out_flat = _sc_apply(slots, cnts, seed16, sub_emb.reshape(-1),
                         etype_pad, n2n_pad, estate_pad,
                         entity_table.reshape(-1), objrel.reshape(-1))
    return out_flat.reshape(N, D)
